# baseline (speedup 1.0000x reference)
import functools

import jax
import jax.numpy as jnp
from jax.experimental import pallas as pl
from jax.experimental.pallas import tpu as pltpu

_BF16 = jnp.bfloat16
_F32 = jnp.float32


def _round_up(n, m):
    return ((n + m - 1) // m) * m


def _enc_kernel(x_ref, m_ref, fe_ref, fbT_ref,
                w1xT_ref, w1f_ref, w1bT_ref, b1T_ref, w2_ref, b2_ref,
                cpart_ref):
    j = pl.program_id(1)

    @pl.when(j == 0)
    def _init():
        cpart_ref[...] = jnp.zeros_like(cpart_ref)

    x = x_ref[0]
    m = m_ref[0]

    flT = jax.lax.dot_general(
        w1f_ref[...], fe_ref[...].astype(_BF16),
        (((0,), (1,)), ((), ())), preferred_element_type=_F32)
    flT = flT + w1bT_ref[...] * fbT_ref[...] + b1T_ref[...]

    h1 = jnp.maximum(x[:, None, :] * w1xT_ref[...][None, :, :]
                     + flT[None, :, :], 0.0)
    h1b = h1.astype(_BF16)

    z = jax.lax.dot_general(
        h1b, w2_ref[...],
        (((1,), (0,)), ((), ())), preferred_element_type=_F32)

    b2 = b2_ref[...]
    s = jnp.sum(jnp.maximum(z + b2[None, :, :], 0.0), axis=1)

    h1u = jnp.maximum(flT, 0.0).astype(_BF16)
    z_u = jax.lax.dot_general(
        h1u, w2_ref[...],
        (((0,), (0,)), ((), ())), preferred_element_type=_F32)
    r_u = jnp.maximum(z_u + b2, 0.0).astype(_BF16)
    m1 = (1.0 - m).astype(_BF16)
    corr = jax.lax.dot_general(
        m1, r_u, (((1,), (0,)), ((), ())), preferred_element_type=_F32)
    cpart_ref[0] += s - corr


def _encoder(x3, m3, fe, fbT, p, tile, steps):
    B = x3.shape[1]
    Hh = p['w1f'].shape[1]
    K = p['w2'].shape[1]

    full = lambda shape: pl.BlockSpec(shape, lambda i, j: (0, 0))

    c_parts = pl.pallas_call(
        _enc_kernel,
        out_shape=jax.ShapeDtypeStruct((2, B, K), _F32),
        grid_spec=pltpu.PrefetchScalarGridSpec(
            num_scalar_prefetch=0,
            grid=(2, steps),
            in_specs=[
                pl.BlockSpec((1, B, tile), lambda i, j: (i * steps + j, 0, 0)),
                pl.BlockSpec((1, B, tile), lambda i, j: (i * steps + j, 0, 0)),
                pl.BlockSpec((tile, K), lambda i, j: (i * steps + j, 0)),
                pl.BlockSpec((1, tile), lambda i, j: (0, i * steps + j)),
                full((Hh, 1)),
                full((K, Hh)),
                full((Hh, 1)),
                full((Hh, 1)),
                full((Hh, K)),
                full((1, K)),
            ],
            out_specs=pl.BlockSpec((1, B, K), lambda i, j: (i, 0, 0)),
        ),
        compiler_params=pltpu.CompilerParams(
            dimension_semantics=("parallel", "arbitrary")),
    )(x3, m3, fe, fbT,
      p['w1x'].T.astype(_BF16), p['w1f'].astype(_BF16),
      p['w1b'].T, p['b1'].T, p['w2'].astype(_BF16), p['b2'])
    return c_parts


def _latent_kernel(cp_ref, wm1_ref, bm1_ref, wm2_ref, bm2_ref,
                   wz1_ref, bz1_ref, wj1z_ref,
                   mu_ref, lv_ref, pzp_ref):
    c = cp_ref[0] + cp_ref[1]
    h = jnp.maximum(
        jnp.dot(c, wm1_ref[...], preferred_element_type=_F32) + bm1_ref[...],
        0.0)
    e = jnp.dot(h, wm2_ref[...], preferred_element_type=_F32) + bm2_ref[...]
    L = e.shape[1] // 2
    mu = e[:, :L]
    mu_ref[...] = mu
    lv_ref[...] = e[:, L:]
    pz = jnp.maximum(
        jnp.dot(mu, wz1_ref[...], preferred_element_type=_F32) + bz1_ref[...],
        0.0)
    pzp_ref[...] = jnp.dot(pz, wj1z_ref[...], preferred_element_type=_F32)


def _latent(c_parts, p):
    B = c_parts.shape[1]
    L = p['wz1'].shape[0]
    Hd = p['wz1'].shape[1]
    mu, lv, pzp = pl.pallas_call(
        _latent_kernel,
        out_shape=(jax.ShapeDtypeStruct((B, L), _F32),
                   jax.ShapeDtypeStruct((B, L), _F32),
                   jax.ShapeDtypeStruct((B, Hd), _F32)),
    )(c_parts, p['wm1'], p['bm1'], p['wm2'], p['bm2'],
      p['wz1'], p['bz1'], p['wj1z'])
    return mu, lv, pzp


def _dec_kernel(pzp_ref, fe_ref, fb_ref,
                wj1f_ref, wj1b_ref, bj1_ref, wj2T_ref, bj2_ref,
                rec_ref):
    fl = jnp.dot(fe_ref[...].astype(_BF16), wj1f_ref[...],
                 preferred_element_type=_F32)
    fl = fl + fb_ref[...] * wj1b_ref[...] + bj1_ref[...]

    j1 = jnp.maximum(pzp_ref[...][:, None, :] + fl[None, :, :], 0.0)
    prod = j1 * wj2T_ref[...][None, :, :]
    rec_ref[...] = jnp.sum(prod, axis=2) + bj2_ref[0, 0]


def _decoder(pzp, fe, fb, p, tile, nsteps):
    B, Hd = pzp.shape
    K = fe.shape[1]
    Dp = fe.shape[0]

    full = lambda shape: pl.BlockSpec(shape, lambda i: (0, 0))

    rec = pl.pallas_call(
        _dec_kernel,
        out_shape=jax.ShapeDtypeStruct((B, Dp), _F32),
        grid_spec=pltpu.PrefetchScalarGridSpec(
            num_scalar_prefetch=0,
            grid=(nsteps,),
            in_specs=[
                full((B, Hd)),
                pl.BlockSpec((tile, K), lambda i: (i, 0)),
                pl.BlockSpec((tile, 1), lambda i: (i, 0)),
                full((K, Hd)),
                full((1, Hd)),
                full((1, Hd)),
                full((1, Hd)),
                full((1, 1)),
            ],
            out_specs=pl.BlockSpec((B, tile), lambda i: (0, i)),
        ),
        compiler_params=pltpu.CompilerParams(
            dimension_semantics=("parallel",)),
    )(pzp, fe, fb,
      p['wj1f'].astype(_BF16), p['wj1b'], p['bj1'], p['wj2'].T, p['bj2'])
    return rec


@functools.partial(jax.jit, static_argnames=("enc_tile", "dec_tile"))
def _forward(x, mask, p, *, enc_tile=512, dec_tile=512):
    B, D = x.shape
    chunk = 2 * enc_tile
    lcm = max(chunk, dec_tile)
    Dp = _round_up(D, lcm)
    fe, fb = p['fe'], p['fb']

    pad = Dp - D
    if pad:
        x = jnp.pad(x, ((0, 0), (0, pad)))
        mask = jnp.pad(mask, ((0, 0), (0, pad)))
        fe = jnp.pad(fe, ((0, pad), (0, 0)))
        fb = jnp.pad(fb, ((0, pad), (0, 0)))

    nt = Dp // enc_tile
    x3 = x.reshape(B, nt, enc_tile).swapaxes(0, 1)
    m3 = mask.reshape(B, nt, enc_tile).swapaxes(0, 1)

    fbT = fb.T
    c_parts = _encoder(x3, m3, fe, fbT, p, enc_tile, Dp // chunk)
    mu, logvar, pzp = _latent(c_parts, p)
    rec = _decoder(pzp, fe, fb, p, dec_tile, Dp // dec_tile)
    return rec[:, :D], mu, logvar


def kernel(x, mask, fe, fb, w1x, w1f, w1b, b1, w2, b2, wm1, bm1, wm2, bm2,
           wz1, bz1, wj1z, wj1f, wj1b, bj1, wj2, bj2):
    p = {
        "fe": fe, "fb": fb, "w1x": w1x, "w1f": w1f, "w1b": w1b, "b1": b1,
        "w2": w2, "b2": b2, "wm1": wm1, "bm1": bm1, "wm2": wm2, "bm2": bm2,
        "wz1": wz1, "bz1": bz1, "wj1z": wj1z, "wj1f": wj1f, "wj1b": wj1b,
        "bj1": bj1, "wj2": wj2, "bj2": bj2,
    }
    return _forward(x, mask, p, enc_tile=512, dec_tile=512)


# speedup vs baseline: 1.0667x; 1.0667x over previous
import functools

import jax
import jax.numpy as jnp
from jax.experimental import pallas as pl
from jax.experimental.pallas import tpu as pltpu

_BF16 = jnp.bfloat16
_F32 = jnp.float32


def _round_up(n, m):
    return ((n + m - 1) // m) * m


def _enc_kernel(x_ref, m_ref, fe_ref, fbT_ref,
                w1xT_ref, w1f_ref, w1bT_ref, b1T_ref, w2_ref, b2_ref,
                cpart_ref):
    j = pl.program_id(1)

    @pl.when(j == 0)
    def _init():
        cpart_ref[...] = jnp.zeros_like(cpart_ref)

    x = x_ref[...]
    m = m_ref[...]

    flT = jax.lax.dot_general(
        w1f_ref[...], fe_ref[...].astype(_BF16),
        (((0,), (1,)), ((), ())), preferred_element_type=_F32)
    flT = flT + w1bT_ref[...] * fbT_ref[...] + b1T_ref[...]

    h1 = jnp.maximum(x[:, None, :] * w1xT_ref[...][None, :, :]
                     + flT[None, :, :], 0.0)
    h1b = h1.astype(_BF16)

    z = jax.lax.dot_general(
        h1b, w2_ref[...],
        (((1,), (0,)), ((), ())), preferred_element_type=_F32)

    b2 = b2_ref[...]
    s = jnp.sum(jnp.maximum(z + b2[None, :, :], 0.0), axis=1)

    h1u = jnp.maximum(flT, 0.0).astype(_BF16)
    z_u = jax.lax.dot_general(
        h1u, w2_ref[...],
        (((0,), (0,)), ((), ())), preferred_element_type=_F32)
    r_u = jnp.maximum(z_u + b2, 0.0).astype(_BF16)
    m1 = (1.0 - m).astype(_BF16)
    corr = jax.lax.dot_general(
        m1, r_u, (((1,), (0,)), ((), ())), preferred_element_type=_F32)
    cpart_ref[0] += s - corr


def _encoder(x, mask, fe, fbT, p, tile, steps):
    B = x.shape[0]
    Hh = p['w1f'].shape[1]
    K = p['w2'].shape[1]

    full = lambda shape: pl.BlockSpec(shape, lambda i, j: (0, 0))

    c_parts = pl.pallas_call(
        _enc_kernel,
        out_shape=jax.ShapeDtypeStruct((2, B, K), _F32),
        grid_spec=pltpu.PrefetchScalarGridSpec(
            num_scalar_prefetch=0,
            grid=(2, steps),
            in_specs=[
                pl.BlockSpec((B, tile), lambda i, j: (0, i * steps + j)),
                pl.BlockSpec((B, tile), lambda i, j: (0, i * steps + j)),
                pl.BlockSpec((tile, K), lambda i, j: (i * steps + j, 0)),
                pl.BlockSpec((1, tile), lambda i, j: (0, i * steps + j)),
                full((Hh, 1)),
                full((K, Hh)),
                full((Hh, 1)),
                full((Hh, 1)),
                full((Hh, K)),
                full((1, K)),
            ],
            out_specs=pl.BlockSpec((1, B, K), lambda i, j: (i, 0, 0)),
        ),
        compiler_params=pltpu.CompilerParams(
            dimension_semantics=("parallel", "arbitrary")),
    )(x, mask, fe, fbT,
      p['w1x'].T.astype(_BF16), p['w1f'].astype(_BF16),
      p['w1b'].T, p['b1'].T, p['w2'].astype(_BF16), p['b2'])
    return c_parts


def _latent_kernel(cp_ref, wm1_ref, bm1_ref, wm2_ref, bm2_ref,
                   wz1_ref, bz1_ref, wj1z_ref,
                   mu_ref, lv_ref, pzp_ref):
    c = cp_ref[0] + cp_ref[1]
    h = jnp.maximum(
        jnp.dot(c, wm1_ref[...], preferred_element_type=_F32) + bm1_ref[...],
        0.0)
    e = jnp.dot(h, wm2_ref[...], preferred_element_type=_F32) + bm2_ref[...]
    L = e.shape[1] // 2
    mu = e[:, :L]
    mu_ref[...] = mu
    lv_ref[...] = e[:, L:]
    pz = jnp.maximum(
        jnp.dot(mu, wz1_ref[...], preferred_element_type=_F32) + bz1_ref[...],
        0.0)
    pzp_ref[...] = jnp.dot(pz, wj1z_ref[...], preferred_element_type=_F32)


def _latent(c_parts, p):
    B = c_parts.shape[1]
    L = p['wz1'].shape[0]
    Hd = p['wz1'].shape[1]
    mu, lv, pzp = pl.pallas_call(
        _latent_kernel,
        out_shape=(jax.ShapeDtypeStruct((B, L), _F32),
                   jax.ShapeDtypeStruct((B, L), _F32),
                   jax.ShapeDtypeStruct((B, Hd), _F32)),
    )(c_parts, p['wm1'], p['bm1'], p['wm2'], p['bm2'],
      p['wz1'], p['bz1'], p['wj1z'])
    return mu, lv, pzp


def _dec_kernel(pzp_ref, fe_ref, fb_ref,
                wj1f_ref, wj1b_ref, bj1_ref, wj2T_ref, bj2_ref,
                rec_ref):
    fl = jnp.dot(fe_ref[...].astype(_BF16), wj1f_ref[...],
                 preferred_element_type=_F32)
    fl = fl + fb_ref[...] * wj1b_ref[...] + bj1_ref[...]

    j1 = jnp.maximum(pzp_ref[...][:, None, :] + fl[None, :, :], 0.0)
    prod = j1 * wj2T_ref[...][None, :, :]
    rec_ref[...] = jnp.sum(prod, axis=2) + bj2_ref[0, 0]


def _decoder(pzp, fe, fb, p, tile, nsteps):
    B, Hd = pzp.shape
    K = fe.shape[1]
    Dp = fe.shape[0]

    full = lambda shape: pl.BlockSpec(shape, lambda i: (0, 0))

    rec = pl.pallas_call(
        _dec_kernel,
        out_shape=jax.ShapeDtypeStruct((B, Dp), _F32),
        grid_spec=pltpu.PrefetchScalarGridSpec(
            num_scalar_prefetch=0,
            grid=(nsteps,),
            in_specs=[
                full((B, Hd)),
                pl.BlockSpec((tile, K), lambda i: (i, 0)),
                pl.BlockSpec((tile, 1), lambda i: (i, 0)),
                full((K, Hd)),
                full((1, Hd)),
                full((1, Hd)),
                full((1, Hd)),
                full((1, 1)),
            ],
            out_specs=pl.BlockSpec((B, tile), lambda i: (0, i)),
        ),
        compiler_params=pltpu.CompilerParams(
            dimension_semantics=("parallel",)),
    )(pzp, fe, fb,
      p['wj1f'].astype(_BF16), p['wj1b'], p['bj1'], p['wj2'].T, p['bj2'])
    return rec


@functools.partial(jax.jit, static_argnames=("enc_tile", "dec_tile"))
def _forward(x, mask, p, *, enc_tile=512, dec_tile=512):
    B, D = x.shape
    chunk = 2 * enc_tile
    lcm = max(chunk, dec_tile)
    Dp = _round_up(D, lcm)
    fe, fb = p['fe'], p['fb']

    pad = Dp - D
    if pad:
        x = jnp.pad(x, ((0, 0), (0, pad)))
        mask = jnp.pad(mask, ((0, 0), (0, pad)))
        fe = jnp.pad(fe, ((0, pad), (0, 0)))
        fb = jnp.pad(fb, ((0, pad), (0, 0)))

    fbT = fb.T
    c_parts = _encoder(x, mask, fe, fbT, p, enc_tile, Dp // chunk)
    mu, logvar, pzp = _latent(c_parts, p)
    rec = _decoder(pzp, fe, fb, p, dec_tile, Dp // dec_tile)
    return rec[:, :D], mu, logvar


def kernel(x, mask, fe, fb, w1x, w1f, w1b, b1, w2, b2, wm1, bm1, wm2, bm2,
           wz1, bz1, wj1z, wj1f, wj1b, bj1, wj2, bj2):
    p = {
        "fe": fe, "fb": fb, "w1x": w1x, "w1f": w1f, "w1b": w1b, "b1": b1,
        "w2": w2, "b2": b2, "wm1": wm1, "bm1": bm1, "wm2": wm2, "bm2": bm2,
        "wz1": wz1, "bz1": bz1, "wj1z": wj1z, "wj1f": wj1f, "wj1b": wj1b,
        "bj1": bj1, "wj2": wj2, "bj2": bj2,
    }
    return _forward(x, mask, p, enc_tile=256, dec_tile=1024)


# speedup vs baseline: 1.0976x; 1.0290x over previous
import functools

import jax
import jax.numpy as jnp
from jax.experimental import pallas as pl
from jax.experimental.pallas import tpu as pltpu

_BF16 = jnp.bfloat16
_F32 = jnp.float32


def _round_up(n, m):
    return ((n + m - 1) // m) * m


def _enc_kernel(x_ref, m_ref, fe_ref, fbT_ref,
                w1xT_ref, w1f_ref, w1bT_ref, b1T_ref, w2_ref, b2_ref,
                cpart_ref):
    j = pl.program_id(1)

    @pl.when(j == 0)
    def _init():
        cpart_ref[...] = jnp.zeros_like(cpart_ref)

    x = x_ref[...]
    m = m_ref[...]

    flT = jax.lax.dot_general(
        w1f_ref[...], fe_ref[...].astype(_BF16),
        (((0,), (1,)), ((), ())), preferred_element_type=_F32)
    flT = flT + w1bT_ref[...] * fbT_ref[...] + b1T_ref[...]

    h1 = jnp.maximum(x[:, None, :] * w1xT_ref[...][None, :, :]
                     + flT[None, :, :], 0.0)
    h1b = h1.astype(_BF16)

    z = jax.lax.dot_general(
        h1b, w2_ref[...],
        (((1,), (0,)), ((), ())), preferred_element_type=_F32)

    b2 = b2_ref[...]
    s = jnp.sum(jnp.maximum(z + b2[None, :, :], 0.0), axis=1)

    h1u = jnp.maximum(flT, 0.0).astype(_BF16)
    z_u = jax.lax.dot_general(
        h1u, w2_ref[...],
        (((0,), (0,)), ((), ())), preferred_element_type=_F32)
    r_u = jnp.maximum(z_u + b2, 0.0)
    r_hi = r_u.astype(_BF16)
    r_lo = (r_u - r_hi.astype(_F32)).astype(_BF16)
    m1 = (1.0 - m).astype(_BF16)
    dims = (((1,), (0,)), ((), ()))
    corr = (jax.lax.dot_general(m1, r_hi, dims, preferred_element_type=_F32)
            + jax.lax.dot_general(m1, r_lo, dims, preferred_element_type=_F32))
    cpart_ref[0] += s - corr


def _encoder(x, mask, fe, fbT, p, tile, steps):
    B = x.shape[0]
    Hh = p['w1f'].shape[1]
    K = p['w2'].shape[1]

    full = lambda shape: pl.BlockSpec(shape, lambda i, j: (0, 0))

    c_parts = pl.pallas_call(
        _enc_kernel,
        out_shape=jax.ShapeDtypeStruct((2, B, K), _F32),
        grid_spec=pltpu.PrefetchScalarGridSpec(
            num_scalar_prefetch=0,
            grid=(2, steps),
            in_specs=[
                pl.BlockSpec((B, tile), lambda i, j: (0, i * steps + j)),
                pl.BlockSpec((B, tile), lambda i, j: (0, i * steps + j)),
                pl.BlockSpec((tile, K), lambda i, j: (i * steps + j, 0)),
                pl.BlockSpec((1, tile), lambda i, j: (0, i * steps + j)),
                full((Hh, 1)),
                full((K, Hh)),
                full((Hh, 1)),
                full((Hh, 1)),
                full((Hh, K)),
                full((1, K)),
            ],
            out_specs=pl.BlockSpec((1, B, K), lambda i, j: (i, 0, 0)),
        ),
        compiler_params=pltpu.CompilerParams(
            dimension_semantics=("parallel", "arbitrary")),
    )(x, mask, fe, fbT,
      p['w1x'].T.astype(_BF16), p['w1f'].astype(_BF16),
      p['w1b'].T, p['b1'].T, p['w2'].astype(_BF16), p['b2'])
    return c_parts


def _latent_kernel(cp_ref, wm1_ref, bm1_ref, wm2_ref, bm2_ref,
                   wz1_ref, bz1_ref, wj1z_ref,
                   mu_ref, lv_ref, pzp_ref):
    c = cp_ref[0] + cp_ref[1]
    h = jnp.maximum(
        jnp.dot(c, wm1_ref[...], preferred_element_type=_F32) + bm1_ref[...],
        0.0)
    e = jnp.dot(h, wm2_ref[...], preferred_element_type=_F32) + bm2_ref[...]
    L = e.shape[1] // 2
    mu = e[:, :L]
    mu_ref[...] = mu
    lv_ref[...] = e[:, L:]
    pz = jnp.maximum(
        jnp.dot(mu, wz1_ref[...], preferred_element_type=_F32) + bz1_ref[...],
        0.0)
    pzp_ref[...] = jnp.dot(pz, wj1z_ref[...], preferred_element_type=_F32)


def _latent(c_parts, p):
    B = c_parts.shape[1]
    L = p['wz1'].shape[0]
    Hd = p['wz1'].shape[1]
    mu, lv, pzp = pl.pallas_call(
        _latent_kernel,
        out_shape=(jax.ShapeDtypeStruct((B, L), _F32),
                   jax.ShapeDtypeStruct((B, L), _F32),
                   jax.ShapeDtypeStruct((B, Hd), _F32)),
    )(c_parts, p['wm1'], p['bm1'], p['wm2'], p['bm2'],
      p['wz1'], p['bz1'], p['wj1z'])
    return mu, lv, pzp


def _dec_kernel(pzp_ref, fe_ref, fb_ref,
                wj1f_ref, wj1b_ref, bj1_ref, wj2T_ref, bj2_ref,
                rec_ref):
    fl = jnp.dot(fe_ref[...].astype(_BF16), wj1f_ref[...],
                 preferred_element_type=_F32)
    fl = fl + fb_ref[...] * wj1b_ref[...] + bj1_ref[...]

    j1 = jnp.maximum(pzp_ref[...][:, None, :] + fl[None, :, :], 0.0)
    prod = j1 * wj2T_ref[...][None, :, :]
    rec_ref[...] = jnp.sum(prod, axis=2) + bj2_ref[0, 0]


def _decoder(pzp, fe, fb, p, tile, nsteps):
    B, Hd = pzp.shape
    K = fe.shape[1]
    Dp = fe.shape[0]

    full = lambda shape: pl.BlockSpec(shape, lambda i: (0, 0))

    rec = pl.pallas_call(
        _dec_kernel,
        out_shape=jax.ShapeDtypeStruct((B, Dp), _F32),
        grid_spec=pltpu.PrefetchScalarGridSpec(
            num_scalar_prefetch=0,
            grid=(nsteps,),
            in_specs=[
                full((B, Hd)),
                pl.BlockSpec((tile, K), lambda i: (i, 0)),
                pl.BlockSpec((tile, 1), lambda i: (i, 0)),
                full((K, Hd)),
                full((1, Hd)),
                full((1, Hd)),
                full((1, Hd)),
                full((1, 1)),
            ],
            out_specs=pl.BlockSpec((B, tile), lambda i: (0, i)),
        ),
        compiler_params=pltpu.CompilerParams(
            dimension_semantics=("parallel",)),
    )(pzp, fe, fb,
      p['wj1f'].astype(_BF16), p['wj1b'], p['bj1'], p['wj2'].T, p['bj2'])
    return rec


@functools.partial(jax.jit, static_argnames=("enc_tile", "dec_tile"))
def _forward(x, mask, p, *, enc_tile=512, dec_tile=512):
    B, D = x.shape
    chunk = 2 * enc_tile
    lcm = max(chunk, dec_tile)
    Dp = _round_up(D, lcm)
    fe, fb = p['fe'], p['fb']

    pad = Dp - D
    if pad:
        x = jnp.pad(x, ((0, 0), (0, pad)))
        mask = jnp.pad(mask, ((0, 0), (0, pad)))
        fe = jnp.pad(fe, ((0, pad), (0, 0)))
        fb = jnp.pad(fb, ((0, pad), (0, 0)))

    fbT = fb.T
    c_parts = _encoder(x, mask, fe, fbT, p, enc_tile, Dp // chunk)
    mu, logvar, pzp = _latent(c_parts, p)
    rec = _decoder(pzp, fe, fb, p, dec_tile, Dp // dec_tile)
    return rec[:, :D], mu, logvar


def kernel(x, mask, fe, fb, w1x, w1f, w1b, b1, w2, b2, wm1, bm1, wm2, bm2,
           wz1, bz1, wj1z, wj1f, wj1b, bj1, wj2, bj2):
    p = {
        "fe": fe, "fb": fb, "w1x": w1x, "w1f": w1f, "w1b": w1b, "b1": b1,
        "w2": w2, "b2": b2, "wm1": wm1, "bm1": bm1, "wm2": wm2, "bm2": bm2,
        "wz1": wz1, "bz1": bz1, "wj1z": wj1z, "wj1f": wj1f, "wj1b": wj1b,
        "bj1": bj1, "wj2": wj2, "bj2": bj2,
    }
    return _forward(x, mask, p, enc_tile=256, dec_tile=1024)


# speedup vs baseline: 1.1038x; 1.0057x over previous
import functools

import jax
import jax.numpy as jnp
from jax.experimental import pallas as pl
from jax.experimental.pallas import tpu as pltpu

_BF16 = jnp.bfloat16
_F32 = jnp.float32


def _round_up(n, m):
    return ((n + m - 1) // m) * m


def _enc_kernel(x_ref, m_ref, fe_ref, fbT_ref,
                w1xT_ref, w1f_ref, w1bT_ref, b1T_ref, w2_ref, b2_ref,
                cpart_ref):
    j = pl.program_id(1)

    @pl.when(j == 0)
    def _init():
        cpart_ref[...] = jnp.zeros_like(cpart_ref)

    x = x_ref[...]
    m = m_ref[...]

    flT = jax.lax.dot_general(
        w1f_ref[...], fe_ref[...].astype(_BF16),
        (((0,), (1,)), ((), ())), preferred_element_type=_F32)
    flT = flT + w1bT_ref[...] * fbT_ref[...] + b1T_ref[...]

    h1 = jnp.maximum(x[:, None, :] * w1xT_ref[...][None, :, :]
                     + flT[None, :, :], 0.0)
    h1b = h1.astype(_BF16)

    z = jax.lax.dot_general(
        h1b, w2_ref[...],
        (((1,), (0,)), ((), ())), preferred_element_type=_F32)

    b2 = b2_ref[...]
    s = jnp.sum(jnp.maximum(z + b2[None, :, :], 0.0), axis=1)

    h1u = jnp.maximum(flT, 0.0).astype(_BF16)
    z_u = jax.lax.dot_general(
        h1u, w2_ref[...],
        (((0,), (0,)), ((), ())), preferred_element_type=_F32)
    r_u = jnp.maximum(z_u + b2, 0.0)
    r_hi = r_u.astype(_BF16)
    r_lo = (r_u - r_hi.astype(_F32)).astype(_BF16)
    m1 = (1.0 - m).astype(_BF16)
    dims = (((1,), (0,)), ((), ()))
    corr = (jax.lax.dot_general(m1, r_hi, dims, preferred_element_type=_F32)
            + jax.lax.dot_general(m1, r_lo, dims, preferred_element_type=_F32))
    cpart_ref[0] += s - corr


def _encoder(x, mask, fe, fbT, p, tile, steps):
    B = x.shape[0]
    Hh = p['w1f'].shape[1]
    K = p['w2'].shape[1]

    full = lambda shape: pl.BlockSpec(shape, lambda i, j: (0, 0))

    c_parts = pl.pallas_call(
        _enc_kernel,
        out_shape=jax.ShapeDtypeStruct((2, B, K), _F32),
        grid_spec=pltpu.PrefetchScalarGridSpec(
            num_scalar_prefetch=0,
            grid=(2, steps),
            in_specs=[
                pl.BlockSpec((B, tile), lambda i, j: (0, i * steps + j)),
                pl.BlockSpec((B, tile), lambda i, j: (0, i * steps + j)),
                pl.BlockSpec((tile, K), lambda i, j: (i * steps + j, 0)),
                pl.BlockSpec((1, tile), lambda i, j: (0, i * steps + j)),
                full((Hh, 1)),
                full((K, Hh)),
                full((Hh, 1)),
                full((Hh, 1)),
                full((Hh, K)),
                full((1, K)),
            ],
            out_specs=pl.BlockSpec((1, B, K), lambda i, j: (i, 0, 0)),
        ),
        compiler_params=pltpu.CompilerParams(
            dimension_semantics=("parallel", "arbitrary")),
    )(x, mask, fe, fbT,
      p['w1x'].T, p['w1f'].astype(_BF16),
      p['w1b'].T, p['b1'].T, p['w2'].astype(_BF16), p['b2'])
    return c_parts


def _latent_kernel(cp_ref, wm1_ref, bm1_ref, wm2_ref, bm2_ref,
                   wz1_ref, bz1_ref, wj1z_ref,
                   mu_ref, lv_ref, pzp_ref):
    c = cp_ref[0] + cp_ref[1]
    h = jnp.maximum(
        jnp.dot(c, wm1_ref[...], preferred_element_type=_F32) + bm1_ref[...],
        0.0)
    e = jnp.dot(h, wm2_ref[...], preferred_element_type=_F32) + bm2_ref[...]
    L = e.shape[1] // 2
    mu = e[:, :L]
    mu_ref[...] = mu
    lv_ref[...] = e[:, L:]
    pz = jnp.maximum(
        jnp.dot(mu, wz1_ref[...], preferred_element_type=_F32) + bz1_ref[...],
        0.0)
    pzp_ref[...] = jnp.dot(pz, wj1z_ref[...], preferred_element_type=_F32)


def _latent(c_parts, p):
    B = c_parts.shape[1]
    L = p['wz1'].shape[0]
    Hd = p['wz1'].shape[1]
    mu, lv, pzp = pl.pallas_call(
        _latent_kernel,
        out_shape=(jax.ShapeDtypeStruct((B, L), _F32),
                   jax.ShapeDtypeStruct((B, L), _F32),
                   jax.ShapeDtypeStruct((B, Hd), _F32)),
    )(c_parts, p['wm1'], p['bm1'], p['wm2'], p['bm2'],
      p['wz1'], p['bz1'], p['wj1z'])
    return mu, lv, pzp


def _dec_kernel(pzp_ref, fe_ref, fb_ref,
                wj1f_ref, wj1b_ref, bj1_ref, wj2T_ref, bj2_ref,
                rec_ref):
    fl = jnp.dot(fe_ref[...].astype(_BF16), wj1f_ref[...],
                 preferred_element_type=_F32)
    fl = fl + fb_ref[...] * wj1b_ref[...] + bj1_ref[...]

    j1 = jnp.maximum(pzp_ref[...][:, None, :] + fl[None, :, :], 0.0)
    prod = j1 * wj2T_ref[...][None, :, :]
    rec_ref[...] = jnp.sum(prod, axis=2) + bj2_ref[0, 0]


def _decoder(pzp, fe, fb, p, tile, nsteps):
    B, Hd = pzp.shape
    K = fe.shape[1]
    Dp = fe.shape[0]

    full = lambda shape: pl.BlockSpec(shape, lambda i: (0, 0))

    rec = pl.pallas_call(
        _dec_kernel,
        out_shape=jax.ShapeDtypeStruct((B, Dp), _F32),
        grid_spec=pltpu.PrefetchScalarGridSpec(
            num_scalar_prefetch=0,
            grid=(nsteps,),
            in_specs=[
                full((B, Hd)),
                pl.BlockSpec((tile, K), lambda i: (i, 0)),
                pl.BlockSpec((tile, 1), lambda i: (i, 0)),
                full((K, Hd)),
                full((1, Hd)),
                full((1, Hd)),
                full((1, Hd)),
                full((1, 1)),
            ],
            out_specs=pl.BlockSpec((B, tile), lambda i: (0, i)),
        ),
        compiler_params=pltpu.CompilerParams(
            dimension_semantics=("parallel",)),
    )(pzp, fe, fb,
      p['wj1f'].astype(_BF16), p['wj1b'], p['bj1'], p['wj2'].T, p['bj2'])
    return rec


@functools.partial(jax.jit, static_argnames=("enc_tile", "dec_tile"))
def _forward(x, mask, p, *, enc_tile=512, dec_tile=512):
    B, D = x.shape
    chunk = 2 * enc_tile
    lcm = max(chunk, dec_tile)
    Dp = _round_up(D, lcm)
    fe, fb = p['fe'], p['fb']

    pad = Dp - D
    if pad:
        x = jnp.pad(x, ((0, 0), (0, pad)))
        mask = jnp.pad(mask, ((0, 0), (0, pad)))
        fe = jnp.pad(fe, ((0, pad), (0, 0)))
        fb = jnp.pad(fb, ((0, pad), (0, 0)))

    fbT = fb.T
    c_parts = _encoder(x, mask, fe, fbT, p, enc_tile, Dp // chunk)
    mu, logvar, pzp = _latent(c_parts, p)
    rec = _decoder(pzp, fe, fb, p, dec_tile, Dp // dec_tile)
    return rec[:, :D], mu, logvar


def kernel(x, mask, fe, fb, w1x, w1f, w1b, b1, w2, b2, wm1, bm1, wm2, bm2,
           wz1, bz1, wj1z, wj1f, wj1b, bj1, wj2, bj2):
    p = {
        "fe": fe, "fb": fb, "w1x": w1x, "w1f": w1f, "w1b": w1b, "b1": b1,
        "w2": w2, "b2": b2, "wm1": wm1, "bm1": bm1, "wm2": wm2, "bm2": bm2,
        "wz1": wz1, "bz1": bz1, "wj1z": wj1z, "wj1f": wj1f, "wj1b": wj1b,
        "bj1": bj1, "wj2": wj2, "bj2": bj2,
    }
    return _forward(x, mask, p, enc_tile=256, dec_tile=1024)
